# speedup vs baseline: 1.7227x; 1.7227x over previous
import jax
import jax.numpy as jnp
from jax.experimental import pallas as pl


_TILE = 512


def _add_body(x_ref, pos_ref, out_ref):
    out_ref[...] = x_ref[...] + pos_ref[...][None, :, :]


def kernel(x, pos_table):
    batch, seq_len, embed_dim = x.shape
    grid = (seq_len // _TILE,)
    return pl.pallas_call(
        _add_body,
        grid=grid,
        in_specs=[
            pl.BlockSpec((batch, _TILE, embed_dim), lambda i: (0, i, 0)),
            pl.BlockSpec((_TILE, embed_dim), lambda i: (i, 0)),
        ],
        out_specs=pl.BlockSpec((batch, _TILE, embed_dim), lambda i: (0, i, 0)),
        out_shape=jax.ShapeDtypeStruct(x.shape, x.dtype),
    )(x, pos_table[:seq_len])
